# baseline (speedup 1.0000x reference)
import functools

import jax
import jax.numpy as jnp
from jax import lax
from jax.experimental import pallas as pl
from jax.experimental.pallas import tpu as pltpu
from jax.experimental.pallas import tpu_sc as plsc

N = 100000
E = 6400000
G = 1024

NC = 2
NS = 16
NW = NC * NS
L = 16

RW = 128
KR = 16
ECH = KR * RW
NECH = E // ECH
EROWS = E // RW

NODE_CH = 1024
NPAD = 100352
NNCH = NPAD // NODE_CH
DUMP = NPAD // NS
BLK = NW * NODE_CH

GP = 1056

_mesh = plsc.VectorSubcoreMesh(
    core_axis_name="c", subcore_axis_name="s", num_cores=NC, num_subcores=NS)
_params = pltpu.CompilerParams(needs_layout_passes=False)
f32 = jnp.float32
i32 = jnp.int32


def _rsqrt(d):
    i = lax.bitcast_convert_type(d, i32)
    i = jnp.int32(0x5F3759DF) - lax.shift_right_logical(i, 1)
    y = lax.bitcast_convert_type(i, f32)
    for _ in range(3):
        y = y * (jnp.float32(1.5) - jnp.float32(0.5) * d * y * y)
    return y


def _zero_vmem(ref, n):
    for v in range(n // L):
        ref[pl.ds(v * L, L)] = jnp.zeros((L,), f32)


def _zero_vmem_big(ref, n):
    def body(it, carry):
        base = it * 256
        for k in range(16):
            ref[pl.ds(base + k * L, L)] = jnp.zeros((L,), f32)
        return carry

    lax.fori_loop(0, n // 256, body, 0)


def _zero_shared(shared, zbuf, sid):
    nz = shared.shape[0] // NODE_CH
    for it in range((nz + NS - 1) // NS):
        c = sid + NS * it

        @pl.when(c < nz)
        def _():
            pltpu.sync_copy(zbuf, shared.at[pl.ds(c * NODE_CH, NODE_CH)])


def _dump_shared(shared, dbuf, hbm, base, sid):
    piece = DUMP // 4
    for k in range(4):
        off = sid * DUMP + k * piece
        pltpu.sync_copy(shared.at[pl.ds(off, piece)], dbuf)
        pltpu.sync_copy(dbuf, hbm.at[pl.ds(base + off, piece)])


@functools.partial(
    pl.kernel,
    out_type=jax.ShapeDtypeStruct((NC * NPAD,), f32),
    mesh=_mesh,
    compiler_params=_params,
    scratch_types=[
        pltpu.VMEM((4, KR, RW), i32),
        pltpu.VMEM((RW,), f32),
        pltpu.VMEM((DUMP // 4,), f32),
        pltpu.VMEM_SHARED((NPAD,), f32),
        pltpu.SemaphoreType.DMA,
        pltpu.SemaphoreType.DMA,
        pltpu.SemaphoreType.DMA,
        pltpu.SemaphoreType.DMA,
        pltpu.SemaphoreType.DMA,
        pltpu.SemaphoreType.DMA,
    ],
)
def _k1(dst_hbm, degp_hbm, dbuf, ones_v, dmpb, shared,
        semi0, semi1, sems0, sems1, sems2, sems3):
    cid = lax.axis_index("c")
    sid = lax.axis_index("s")
    wid = sid * NC + cid
    semi = (semi0, semi1)
    sems = (sems0, sems1, sems2, sems3)

    _zero_vmem(dmpb, NODE_CH)
    for v in range(RW // L):
        ones_v[pl.ds(v * L, L)] = jnp.ones((L,), f32)
    _zero_shared(shared, dmpb.at[pl.ds(0, NODE_CH)], sid)
    plsc.subcore_barrier()

    def start_in(i, s4):
        c = wid + NW * i

        @pl.when(c < NECH)
        def _():
            pltpu.async_copy(dst_hbm.at[pl.ds(c * KR, KR)], dbuf.at[s4],
                             semi[s4 % 2])

    def wait_in(s4):
        pltpu.make_async_copy(dst_hbm.at[pl.ds(0, KR)], dbuf.at[s4],
                              semi[s4 % 2]).wait()

    def drain_scatter(s4):
        for j in range(KR):
            pltpu.make_async_copy(degp_hbm.at[pl.ds(0, RW)], ones_v,
                                  sems[s4]).wait()

    start_in(0, 0)

    def body(it, carry):
        for b in range(4):
            i = 4 * it + b
            c = wid + NW * i
            start_in(i + 1, (b + 1) % 4)

            @pl.when(c < NECH)
            def _():
                wait_in(b)

                @pl.when(i >= 2)
                def _():
                    drain_scatter((b + 2) % 4)

                for j in range(KR):
                    pltpu.async_copy(ones_v, shared.at[dbuf.at[b, j]],
                                     sems[b], add=True)

        return carry

    nit = (NECH + NW - 1) // NW
    lax.fori_loop(0, (nit + 3) // 4, body, 0)
    i_last = (NECH - 1 - wid) // NW
    for s in range(4):
        @pl.when(jnp.logical_or(i_last % 4 == s, (i_last - 1) % 4 == s))
        def _():
            drain_scatter(s)

    plsc.subcore_barrier()
    _dump_shared(shared, dmpb, degp_hbm, cid * NPAD, sid)


ROWS = NPAD // RW
CHA = 512
NACH = NPAD // CHA
CROW = CHA // RW


@functools.partial(
    pl.kernel,
    out_type=(
        jax.ShapeDtypeStruct((NC * NPAD,), f32),
        jax.ShapeDtypeStruct((NC * NPAD,), f32),
        jax.ShapeDtypeStruct((NPAD,), f32),
        jax.ShapeDtypeStruct((NPAD,), f32),
    ),
    mesh=_mesh,
    compiler_params=_params,
    scratch_types=[
        pltpu.VMEM((NPAD,), f32),
        pltpu.VMEM((2, KR, RW), i32),
        pltpu.VMEM((4, KR, RW), i32),
        pltpu.VMEM((4, KR, RW), f32),
        pltpu.VMEM((DUMP // 4,), f32),
        pltpu.VMEM((16,), f32),
        pltpu.VMEM_SHARED((NPAD,), f32),
        pltpu.SemaphoreType.DMA,
        pltpu.SemaphoreType.DMA,
        pltpu.SemaphoreType.DMA,
        pltpu.SemaphoreType.DMA,
        pltpu.SemaphoreType.DMA,
        pltpu.SemaphoreType.DMA,
    ],
)
def _k3(src_hbm, dst_hbm, x0h, x1h, x2h, x3h, degp_h, wb_h,
        accp_hbm, zscr_hbm, n_hbm, p_hbm,
        ztab, sbuf, dbuf, vbuf, dmpb, wbuf, shared,
        semi0, semi1, sems0, sems1, sems2, sems3):
    cid = lax.axis_index("c")
    sid = lax.axis_index("s")
    wid = sid * NC + cid
    semi = (semi0, semi1)
    sems = (sems0, sems1, sems2, sems3)

    pltpu.sync_copy(wb_h, wbuf)
    wv = wbuf[pl.ds(0, L)]
    w0 = wv[0]
    w1 = wv[1]
    w2 = wv[2]
    w3 = wv[3]

    _zero_vmem(dmpb, NODE_CH)
    _zero_shared(shared, dmpb.at[pl.ds(0, NODE_CH)], sid)

    def abody(k, carry):
        c = sid + NS * k

        @pl.when(c < NACH)
        def _():
            r0 = c * CROW
            pltpu.async_copy(x0h.at[pl.ds(r0, CROW)],
                             sbuf.at[0, pl.ds(0, CROW)], semi0)
            pltpu.async_copy(x1h.at[pl.ds(r0, CROW)],
                             sbuf.at[0, pl.ds(CROW, CROW)], semi0)
            pltpu.async_copy(x2h.at[pl.ds(r0, CROW)],
                             sbuf.at[1, pl.ds(0, CROW)], semi0)
            pltpu.async_copy(x3h.at[pl.ds(r0, CROW)],
                             sbuf.at[1, pl.ds(CROW, CROW)], semi0)
            pltpu.async_copy(degp_h.at[pl.ds(r0, CROW)],
                             dbuf.at[0, pl.ds(0, CROW)], semi0)
            pltpu.async_copy(degp_h.at[pl.ds(ROWS + r0, CROW)],
                             dbuf.at[0, pl.ds(CROW, CROW)], semi0)
            for _ in range(6):
                pltpu.make_async_copy(x0h.at[pl.ds(0, CROW)],
                                      sbuf.at[0, pl.ds(0, CROW)],
                                      semi0).wait()
            for v in range(CHA // L):
                r, co = divmod(v, RW // L)
                co *= L
                sl = pl.ds(co, L)
                x0v = lax.bitcast_convert_type(sbuf[0, r, sl], f32)
                x1v = lax.bitcast_convert_type(sbuf[0, r + CROW, sl], f32)
                x2v = lax.bitcast_convert_type(sbuf[1, r, sl], f32)
                x3v = lax.bitcast_convert_type(sbuf[1, r + CROW, sl], f32)
                d0 = lax.bitcast_convert_type(dbuf[0, r, sl], f32)
                d1 = lax.bitcast_convert_type(dbuf[0, r + CROW, sl], f32)
                p = x0v * w0 + x1v * w1 + x2v * w2 + x3v * w3
                d = jnp.maximum(d0 + d1 + jnp.float32(1.0), jnp.float32(1.0))
                y = _rsqrt(d)
                dmpb[pl.ds(v * L, L)] = p * y
                dmpb[pl.ds(CHA + v * L, L)] = y
                dmpb[pl.ds(2 * CHA + v * L, L)] = p
            base = c * CHA
            pltpu.sync_copy(dmpb.at[pl.ds(0, CHA)],
                            zscr_hbm.at[pl.ds(cid * NPAD + base, CHA)])

            @pl.when(cid == 0)
            def _():
                pltpu.sync_copy(dmpb.at[pl.ds(CHA, CHA)],
                                n_hbm.at[pl.ds(base, CHA)])
                pltpu.sync_copy(dmpb.at[pl.ds(2 * CHA, CHA)],
                                p_hbm.at[pl.ds(base, CHA)])

        return carry

    lax.fori_loop(0, (NACH + NS - 1) // NS, abody, 0)
    plsc.subcore_barrier()
    pltpu.sync_copy(zscr_hbm.at[pl.ds(cid * NPAD, NPAD)], ztab)

    def start_in(i, s2, s4):
        c = wid + NW * i

        @pl.when(c < NECH)
        def _():
            pltpu.async_copy(src_hbm.at[pl.ds(c * KR, KR)], sbuf.at[s2],
                             semi[s2])
            pltpu.async_copy(dst_hbm.at[pl.ds(c * KR, KR)], dbuf.at[s4],
                             semi[s2])

    def wait_in(s2, s4):
        pltpu.make_async_copy(src_hbm.at[pl.ds(0, KR)], sbuf.at[s2],
                              semi[s2]).wait()
        pltpu.make_async_copy(dst_hbm.at[pl.ds(0, KR)], dbuf.at[s4],
                              semi[s2]).wait()

    def drain_scatter(s4):
        for j in range(KR):
            pltpu.make_async_copy(zscr_hbm.at[pl.ds(0, RW)], vbuf.at[s4, j],
                                  sems[s4]).wait()

    start_in(0, 0, 0)

    def body(it, carry):
        for b in range(4):
            i = 4 * it + b
            c = wid + NW * i
            start_in(i + 1, (b + 1) % 2, (b + 1) % 4)

            @pl.when(c < NECH)
            def _():
                wait_in(b % 2, b)
                for j in range(KR):
                    for g in range(RW // L):
                        idx = sbuf[b % 2, j, pl.ds(g * L, L)]
                        vbuf[b, j, pl.ds(g * L, L)] = plsc.load_gather(
                            ztab, [idx])

                @pl.when(i >= 2)
                def _():
                    drain_scatter((b + 2) % 4)

                for j in range(KR):
                    pltpu.async_copy(vbuf.at[b, j],
                                     shared.at[dbuf.at[b, j]],
                                     sems[b], add=True)

        return carry

    nit = (NECH + NW - 1) // NW
    lax.fori_loop(0, (nit + 3) // 4, body, 0)
    i_last = (NECH - 1 - wid) // NW
    for s in range(4):
        @pl.when(jnp.logical_or(i_last % 4 == s, (i_last - 1) % 4 == s))
        def _():
            drain_scatter(s)

    plsc.subcore_barrier()
    _dump_shared(shared, dmpb, accp_hbm, cid * NPAD, sid)


@functools.partial(
    pl.kernel,
    out_type=(
        jax.ShapeDtypeStruct((NW * GP,), f32),
        jax.ShapeDtypeStruct((NW * GP,), f32),
    ),
    mesh=_mesh,
    compiler_params=_params,
    scratch_types=[
        pltpu.VMEM((NODE_CH,), f32),
        pltpu.VMEM((NODE_CH,), f32),
        pltpu.VMEM((NODE_CH,), f32),
        pltpu.VMEM((NODE_CH,), f32),
        pltpu.VMEM((NODE_CH,), i32),
        pltpu.VMEM((16,), f32),
        pltpu.VMEM((GP,), f32),
        pltpu.VMEM((GP,), f32),
        pltpu.SemaphoreType.DMA,
    ],
)
def _k4(accp_h, nrm_h, p_h, batch_h, wb_h, sump_h, cntp_h,
        a0b, a1b, nb, pb, bbuf, wbuf, sumb, cntb, semi):
    cid = lax.axis_index("c")
    sid = lax.axis_index("s")
    wid = sid * NC + cid
    ones = jnp.ones((L,), f32)

    pltpu.sync_copy(wb_h, wbuf)
    bconst = wbuf[pl.ds(0, L)][4]
    _zero_vmem(sumb, GP)
    _zero_vmem(cntb, GP)

    def body(it, carry):
        c = wid + NW * it

        @pl.when(c < NNCH)
        def _():
            base = c * NODE_CH
            pltpu.async_copy(accp_h.at[pl.ds(base, NODE_CH)], a0b, semi)
            pltpu.async_copy(accp_h.at[pl.ds(NPAD + base, NODE_CH)], a1b,
                             semi)
            pltpu.async_copy(nrm_h.at[pl.ds(base, NODE_CH)], nb, semi)
            pltpu.async_copy(p_h.at[pl.ds(base, NODE_CH)], pb, semi)
            pltpu.async_copy(batch_h.at[pl.ds(base, NODE_CH)], bbuf, semi)
            for _ in range(4):
                pltpu.make_async_copy(accp_h.at[pl.ds(0, NODE_CH)], a0b,
                                      semi).wait()
            pltpu.make_async_copy(batch_h.at[pl.ds(0, NODE_CH)], bbuf,
                                  semi).wait()

            def inner(v, carry2):
                sl = pl.ds(v * L, L)
                y = nb[sl]
                o = y * (a0b[sl] + a1b[sl]) + pb[sl] * y * y + bconst
                bi = bbuf[sl]
                plsc.addupdate_scatter(sumb, [bi], o)
                plsc.addupdate_scatter(cntb, [bi], ones)
                return carry2

            lax.fori_loop(0, NODE_CH // L, inner, 0)

        return carry

    lax.fori_loop(0, (NNCH + NW - 1) // NW, body, 0)
    pltpu.sync_copy(sumb, sump_h.at[pl.ds(wid * GP, GP)])
    pltpu.sync_copy(cntb, cntp_h.at[pl.ds(wid * GP, GP)])


@functools.partial(
    pl.kernel,
    out_type=jax.ShapeDtypeStruct((G,), f32),
    mesh=_mesh,
    compiler_params=_params,
    scratch_types=[
        pltpu.VMEM((NW * GP,), f32),
        pltpu.VMEM((NW * GP,), f32),
        pltpu.VMEM((2 * L,), f32),
    ],
)
def _k5(sump_h, cntp_h, logits_h, sbuf, cbuf, obuf):
    cid = lax.axis_index("c")
    sid = lax.axis_index("s")
    wid = sid * NC + cid
    per_w = G // NW

    pltpu.sync_copy(sump_h, sbuf)
    pltpu.sync_copy(cntp_h, cbuf)
    base = wid * per_w
    for v in range(per_w // L):
        s = jnp.zeros((L,), f32)
        cnt = jnp.zeros((L,), f32)
        for w in range(NW):
            s = s + sbuf[pl.ds(w * GP + base + v * L, L)]
            cnt = cnt + cbuf[pl.ds(w * GP + base + v * L, L)]
        obuf[pl.ds(v * L, L)] = s / jnp.maximum(cnt, jnp.float32(1.0))
    pltpu.sync_copy(obuf, logits_h.at[pl.ds(base, per_w)])


def kernel(x, edge_index, batch, W, b):
    src2 = edge_index[0].reshape(EROWS, RW)
    dst2 = edge_index[1].reshape(EROWS, RW)
    xp = jnp.pad(x, ((0, NPAD - N), (0, 0)))
    x0, x1, x2, x3 = (xp[:, j] for j in range(4))
    batchp = jnp.pad(batch, (0, NPAD - N), constant_values=G)
    wb = jnp.zeros((16,), f32).at[:4].set(W[:, 0]).at[4].set(b[0])

    x0i, x1i, x2i, x3i = (
        lax.bitcast_convert_type(col, i32).reshape(ROWS, RW)
        for col in (x0, x1, x2, x3))
    degp = _k1(dst2)
    degp_i = lax.bitcast_convert_type(degp, i32).reshape(NC * ROWS, RW)
    accp, _zs, nrm, p = _k3(src2, dst2, x0i, x1i, x2i, x3i, degp_i, wb)
    sump, cntp = _k4(accp, nrm, p, batchp, wb)
    return _k5(sump, cntp)


# speedup vs baseline: 1.0231x; 1.0231x over previous
import functools

import jax
import jax.numpy as jnp
from jax import lax
from jax.experimental import pallas as pl
from jax.experimental.pallas import tpu as pltpu
from jax.experimental.pallas import tpu_sc as plsc

N = 100000
E = 6400000
G = 1024

NC = 2
NS = 16
NW = NC * NS
L = 16

RW = 128
KR = 16
ECH = KR * RW
NECH = E // ECH
EROWS = E // RW

NODE_CH = 1024
NPAD = 100352
NNCH = NPAD // NODE_CH
DUMP = NPAD // NS
BLK = NW * NODE_CH

GP = 1056

_mesh = plsc.VectorSubcoreMesh(
    core_axis_name="c", subcore_axis_name="s", num_cores=NC, num_subcores=NS)
_params = pltpu.CompilerParams(needs_layout_passes=False)
f32 = jnp.float32
i32 = jnp.int32


def _rsqrt(d):
    i = lax.bitcast_convert_type(d, i32)
    i = jnp.int32(0x5F3759DF) - lax.shift_right_logical(i, 1)
    y = lax.bitcast_convert_type(i, f32)
    for _ in range(3):
        y = y * (jnp.float32(1.5) - jnp.float32(0.5) * d * y * y)
    return y


def _zero_vmem(ref, n):
    for v in range(n // L):
        ref[pl.ds(v * L, L)] = jnp.zeros((L,), f32)


def _zero_vmem_big(ref, n):
    def body(it, carry):
        base = it * 256
        for k in range(16):
            ref[pl.ds(base + k * L, L)] = jnp.zeros((L,), f32)
        return carry

    lax.fori_loop(0, n // 256, body, 0)


def _zero_shared(shared, zbuf, sid):
    nz = shared.shape[0] // NODE_CH
    for it in range((nz + NS - 1) // NS):
        c = sid + NS * it

        @pl.when(c < nz)
        def _():
            pltpu.sync_copy(zbuf, shared.at[pl.ds(c * NODE_CH, NODE_CH)])


def _dump_shared(shared, dbuf, hbm, base, sid):
    half = DUMP // 2
    for k in range(2):
        off = sid * DUMP + k * half
        pltpu.sync_copy(shared.at[pl.ds(off, half)], dbuf)
        pltpu.sync_copy(dbuf, hbm.at[pl.ds(base + off, half)])


@functools.partial(
    pl.kernel,
    out_type=jax.ShapeDtypeStruct((NC * NPAD,), f32),
    mesh=_mesh,
    compiler_params=_params,
    scratch_types=[
        pltpu.VMEM((4, KR, RW), i32),
        pltpu.VMEM((RW,), f32),
        pltpu.VMEM((DUMP // 2,), f32),
        pltpu.VMEM_SHARED((NPAD,), f32),
        pltpu.SemaphoreType.DMA,
        pltpu.SemaphoreType.DMA,
        pltpu.SemaphoreType.DMA,
        pltpu.SemaphoreType.DMA,
        pltpu.SemaphoreType.DMA,
        pltpu.SemaphoreType.DMA,
    ],
)
def _k1(dst_hbm, degp_hbm, dbuf, ones_v, dmpb, shared,
        semi0, semi1, sems0, sems1, sems2, sems3):
    cid = lax.axis_index("c")
    sid = lax.axis_index("s")
    wid = sid * NC + cid
    semi = (semi0, semi1)
    sems = (sems0, sems1, sems2, sems3)

    _zero_vmem(dmpb, NODE_CH)
    for v in range(RW // L):
        ones_v[pl.ds(v * L, L)] = jnp.ones((L,), f32)
    _zero_shared(shared, dmpb.at[pl.ds(0, NODE_CH)], sid)
    plsc.subcore_barrier()

    def start_in(i, s4):
        c = wid + NW * i

        @pl.when(c < NECH)
        def _():
            pltpu.async_copy(dst_hbm.at[pl.ds(c * KR, KR)], dbuf.at[s4],
                             semi[s4 % 2])

    def wait_in(s4):
        pltpu.make_async_copy(dst_hbm.at[pl.ds(0, KR)], dbuf.at[s4],
                              semi[s4 % 2]).wait()

    def drain_scatter(s4):
        for j in range(KR):
            pltpu.make_async_copy(degp_hbm.at[pl.ds(0, RW)], ones_v,
                                  sems[s4]).wait()

    start_in(0, 0)

    def body(it, carry):
        for b in range(4):
            i = 4 * it + b
            c = wid + NW * i
            start_in(i + 1, (b + 1) % 4)

            @pl.when(c < NECH)
            def _():
                wait_in(b)

                @pl.when(i >= 2)
                def _():
                    drain_scatter((b + 2) % 4)

                for j in range(KR):
                    pltpu.async_copy(ones_v, shared.at[dbuf.at[b, j]],
                                     sems[b], add=True)

        return carry

    nit = (NECH + NW - 1) // NW
    lax.fori_loop(0, (nit + 3) // 4, body, 0)
    i_last = (NECH - 1 - wid) // NW
    for s in range(4):
        @pl.when(jnp.logical_or(i_last % 4 == s, (i_last - 1) % 4 == s))
        def _():
            drain_scatter(s)

    plsc.subcore_barrier()
    _dump_shared(shared, dmpb, degp_hbm, cid * NPAD, sid)


@functools.partial(
    pl.kernel,
    out_type=(
        jax.ShapeDtypeStruct((NPAD,), f32),
        jax.ShapeDtypeStruct((NPAD,), f32),
        jax.ShapeDtypeStruct((NPAD,), f32),
    ),
    mesh=_mesh,
    compiler_params=_params,
    scratch_types=[
        pltpu.VMEM((NODE_CH,), f32),
        pltpu.VMEM((NODE_CH,), f32),
        pltpu.VMEM((NODE_CH,), f32),
        pltpu.VMEM((NODE_CH,), f32),
        pltpu.VMEM((NODE_CH,), f32),
        pltpu.VMEM((NODE_CH,), f32),
        pltpu.VMEM((16,), f32),
        pltpu.VMEM((NODE_CH,), f32),
        pltpu.VMEM((NODE_CH,), f32),
        pltpu.VMEM((NODE_CH,), f32),
        pltpu.SemaphoreType.DMA,
    ],
)
def _k2(x0h, x1h, x2h, x3h, degp_h, wb_h, z_h, n_h, p_h,
        x0b, x1b, x2b, x3b, d0b, d1b, wbuf, zb, nb, pb, semi):
    cid = lax.axis_index("c")
    sid = lax.axis_index("s")
    wid = sid * NC + cid

    pltpu.sync_copy(wb_h, wbuf)
    wv = wbuf[pl.ds(0, L)]
    w0 = wv[0]
    w1 = wv[1]
    w2 = wv[2]
    w3 = wv[3]

    def body(it, carry):
        c = wid + NW * it

        @pl.when(c < NNCH)
        def _():
            base = c * NODE_CH
            pltpu.async_copy(x0h.at[pl.ds(base, NODE_CH)], x0b, semi)
            pltpu.async_copy(x1h.at[pl.ds(base, NODE_CH)], x1b, semi)
            pltpu.async_copy(x2h.at[pl.ds(base, NODE_CH)], x2b, semi)
            pltpu.async_copy(x3h.at[pl.ds(base, NODE_CH)], x3b, semi)
            pltpu.async_copy(degp_h.at[pl.ds(base, NODE_CH)], d0b, semi)
            pltpu.async_copy(degp_h.at[pl.ds(NPAD + base, NODE_CH)], d1b,
                             semi)
            for _ in range(6):
                pltpu.make_async_copy(x0h.at[pl.ds(0, NODE_CH)], x0b,
                                      semi).wait()

            def inner(v, carry2):
                sl = pl.ds(v * L, L)
                p = (x0b[sl] * w0 + x1b[sl] * w1
                     + x2b[sl] * w2 + x3b[sl] * w3)
                d = d0b[sl] + d1b[sl] + jnp.float32(1.0)
                d = jnp.maximum(d, jnp.float32(1.0))
                y = _rsqrt(d)
                nb[sl] = y
                pb[sl] = p
                zb[sl] = p * y
                return carry2

            lax.fori_loop(0, NODE_CH // L, inner, 0)
            pltpu.sync_copy(zb, z_h.at[pl.ds(base, NODE_CH)])
            pltpu.sync_copy(nb, n_h.at[pl.ds(base, NODE_CH)])
            pltpu.sync_copy(pb, p_h.at[pl.ds(base, NODE_CH)])

        return carry

    lax.fori_loop(0, (NNCH + NW - 1) // NW, body, 0)


@functools.partial(
    pl.kernel,
    out_type=jax.ShapeDtypeStruct((NC * NPAD,), f32),
    mesh=_mesh,
    compiler_params=_params,
    scratch_types=[
        pltpu.VMEM((NPAD,), f32),
        pltpu.VMEM((2, KR, RW), i32),
        pltpu.VMEM((4, KR, RW), i32),
        pltpu.VMEM((4, KR, RW), f32),
        pltpu.VMEM((DUMP // 2,), f32),
        pltpu.VMEM_SHARED((NPAD,), f32),
        pltpu.SemaphoreType.DMA,
        pltpu.SemaphoreType.DMA,
        pltpu.SemaphoreType.DMA,
        pltpu.SemaphoreType.DMA,
        pltpu.SemaphoreType.DMA,
        pltpu.SemaphoreType.DMA,
    ],
)
def _k3(src_hbm, dst_hbm, z_hbm, accp_hbm,
        ztab, sbuf, dbuf, vbuf, dmpb, shared,
        semi0, semi1, sems0, sems1, sems2, sems3):
    cid = lax.axis_index("c")
    sid = lax.axis_index("s")
    wid = sid * NC + cid
    semi = (semi0, semi1)
    sems = (sems0, sems1, sems2, sems3)

    _zero_vmem(dmpb, NODE_CH)
    _zero_shared(shared, dmpb.at[pl.ds(0, NODE_CH)], sid)
    pltpu.sync_copy(z_hbm, ztab)
    plsc.subcore_barrier()

    def start_in(i, s2, s4):
        c = wid + NW * i

        @pl.when(c < NECH)
        def _():
            pltpu.async_copy(src_hbm.at[pl.ds(c * KR, KR)], sbuf.at[s2],
                             semi[s2])
            pltpu.async_copy(dst_hbm.at[pl.ds(c * KR, KR)], dbuf.at[s4],
                             semi[s2])

    def wait_in(s2, s4):
        pltpu.make_async_copy(src_hbm.at[pl.ds(0, KR)], sbuf.at[s2],
                              semi[s2]).wait()
        pltpu.make_async_copy(dst_hbm.at[pl.ds(0, KR)], dbuf.at[s4],
                              semi[s2]).wait()

    def drain_scatter(s4):
        for j in range(KR):
            pltpu.make_async_copy(z_hbm.at[pl.ds(0, RW)], vbuf.at[s4, j],
                                  sems[s4]).wait()

    start_in(0, 0, 0)

    def body(it, carry):
        for b in range(4):
            i = 4 * it + b
            c = wid + NW * i
            start_in(i + 1, (b + 1) % 2, (b + 1) % 4)

            @pl.when(c < NECH)
            def _():
                wait_in(b % 2, b)
                for j in range(KR):
                    for g in range(RW // L):
                        idx = sbuf[b % 2, j, pl.ds(g * L, L)]
                        vbuf[b, j, pl.ds(g * L, L)] = plsc.load_gather(
                            ztab, [idx])

                @pl.when(i >= 2)
                def _():
                    drain_scatter((b + 2) % 4)

                for j in range(KR):
                    pltpu.async_copy(vbuf.at[b, j],
                                     shared.at[dbuf.at[b, j]],
                                     sems[b], add=True)

        return carry

    nit = (NECH + NW - 1) // NW
    lax.fori_loop(0, (nit + 3) // 4, body, 0)
    i_last = (NECH - 1 - wid) // NW
    for s in range(4):
        @pl.when(jnp.logical_or(i_last % 4 == s, (i_last - 1) % 4 == s))
        def _():
            drain_scatter(s)

    plsc.subcore_barrier()
    _dump_shared(shared, dmpb, accp_hbm, cid * NPAD, sid)


@functools.partial(
    pl.kernel,
    out_type=(
        jax.ShapeDtypeStruct((NW * GP,), f32),
        jax.ShapeDtypeStruct((NW * GP,), f32),
    ),
    mesh=_mesh,
    compiler_params=_params,
    scratch_types=[
        pltpu.VMEM((NODE_CH,), f32),
        pltpu.VMEM((NODE_CH,), f32),
        pltpu.VMEM((NODE_CH,), f32),
        pltpu.VMEM((NODE_CH,), f32),
        pltpu.VMEM((NODE_CH,), i32),
        pltpu.VMEM((16,), f32),
        pltpu.VMEM((GP,), f32),
        pltpu.VMEM((GP,), f32),
        pltpu.SemaphoreType.DMA,
    ],
)
def _k4(accp_h, nrm_h, p_h, batch_h, wb_h, sump_h, cntp_h,
        a0b, a1b, nb, pb, bbuf, wbuf, sumb, cntb, semi):
    cid = lax.axis_index("c")
    sid = lax.axis_index("s")
    wid = sid * NC + cid
    ones = jnp.ones((L,), f32)

    pltpu.sync_copy(wb_h, wbuf)
    bconst = wbuf[pl.ds(0, L)][4]
    _zero_vmem(sumb, GP)
    _zero_vmem(cntb, GP)

    def body(it, carry):
        c = wid + NW * it

        @pl.when(c < NNCH)
        def _():
            base = c * NODE_CH
            pltpu.async_copy(accp_h.at[pl.ds(base, NODE_CH)], a0b, semi)
            pltpu.async_copy(accp_h.at[pl.ds(NPAD + base, NODE_CH)], a1b,
                             semi)
            pltpu.async_copy(nrm_h.at[pl.ds(base, NODE_CH)], nb, semi)
            pltpu.async_copy(p_h.at[pl.ds(base, NODE_CH)], pb, semi)
            pltpu.async_copy(batch_h.at[pl.ds(base, NODE_CH)], bbuf, semi)
            for _ in range(4):
                pltpu.make_async_copy(accp_h.at[pl.ds(0, NODE_CH)], a0b,
                                      semi).wait()
            pltpu.make_async_copy(batch_h.at[pl.ds(0, NODE_CH)], bbuf,
                                  semi).wait()

            def inner(v, carry2):
                sl = pl.ds(v * L, L)
                y = nb[sl]
                o = y * (a0b[sl] + a1b[sl]) + pb[sl] * y * y + bconst
                bi = bbuf[sl]
                plsc.addupdate_scatter(sumb, [bi], o)
                plsc.addupdate_scatter(cntb, [bi], ones)
                return carry2

            lax.fori_loop(0, NODE_CH // L, inner, 0)

        return carry

    lax.fori_loop(0, (NNCH + NW - 1) // NW, body, 0)
    pltpu.sync_copy(sumb, sump_h.at[pl.ds(wid * GP, GP)])
    pltpu.sync_copy(cntb, cntp_h.at[pl.ds(wid * GP, GP)])


@functools.partial(
    pl.kernel,
    out_type=jax.ShapeDtypeStruct((G,), f32),
    mesh=_mesh,
    compiler_params=_params,
    scratch_types=[
        pltpu.VMEM((NW * GP,), f32),
        pltpu.VMEM((NW * GP,), f32),
        pltpu.VMEM((2 * L,), f32),
    ],
)
def _k5(sump_h, cntp_h, logits_h, sbuf, cbuf, obuf):
    cid = lax.axis_index("c")
    sid = lax.axis_index("s")
    wid = sid * NC + cid
    per_w = G // NW

    pltpu.sync_copy(sump_h, sbuf)
    pltpu.sync_copy(cntp_h, cbuf)
    base = wid * per_w
    for v in range(per_w // L):
        s = jnp.zeros((L,), f32)
        cnt = jnp.zeros((L,), f32)
        for w in range(NW):
            s = s + sbuf[pl.ds(w * GP + base + v * L, L)]
            cnt = cnt + cbuf[pl.ds(w * GP + base + v * L, L)]
        obuf[pl.ds(v * L, L)] = s / jnp.maximum(cnt, jnp.float32(1.0))
    pltpu.sync_copy(obuf, logits_h.at[pl.ds(base, per_w)])


def kernel(x, edge_index, batch, W, b):
    src2 = edge_index[0].reshape(EROWS, RW)
    dst2 = edge_index[1].reshape(EROWS, RW)
    xp = jnp.pad(x, ((0, NPAD - N), (0, 0)))
    x0, x1, x2, x3 = (xp[:, j] for j in range(4))
    batchp = jnp.pad(batch, (0, NPAD - N), constant_values=G)
    wb = jnp.zeros((16,), f32).at[:4].set(W[:, 0]).at[4].set(b[0])

    degp = _k1(dst2)
    z, nrm, p = _k2(x0, x1, x2, x3, degp, wb)
    accp = _k3(src2, dst2, z)
    sump, cntp = _k4(accp, nrm, p, batchp, wb)
    return _k5(sump, cntp)
